# baseline (speedup 1.0000x reference)
import functools

import jax
import jax.numpy as jnp
from jax import lax
from jax.experimental import pallas as pl
from jax.experimental.pallas import tpu as pltpu
from jax.experimental.pallas import tpu_sc as plsc

N = 10000
D = 256
H = 512
E = 160000

NP = 10240
FC = 4
CW = 128
BN = 512
NT = NP // BN
ET = 16
EB = 64
NB = 160
EP = ET * NB * EB
STRIPE = NP // ET
JC = FC // 2


def _ln(x, g, b):
    m = jnp.mean(x, axis=-1, keepdims=True)
    xc = x - m
    v = jnp.mean(xc * xc, axis=-1, keepdims=True)
    return xc * lax.rsqrt(v + 1e-5) * g + b


def _mlp_body(emb_ref, nf_ref, w0_ref, b0_ref, g0_ref, bb0_ref,
              w1_ref, b1_ref, g1_ref, bb1_ref,
              w2_ref, b2_ref, g2_ref, bb2_ref,
              wa_ref, ba_ref, ht_ref, a_ref):
    h = emb_ref[...] + nf_ref[...]
    for w_ref, b_ref, g_ref, bb_ref in (
        (w0_ref, b0_ref, g0_ref, bb0_ref),
        (w1_ref, b1_ref, g1_ref, bb1_ref),
        (w2_ref, b2_ref, g2_ref, bb2_ref),
    ):
        x = lax.dot_general(h, w_ref[...], (((1,), (1,)), ((), ())),
                            preferred_element_type=jnp.float32)
        x = _ln(x + b_ref[...], g_ref[...], bb_ref[...])
        h = jnp.maximum(x, 0.0)
    a = lax.dot_general(wa_ref[...], h, (((1,), (1,)), ((), ())),
                        preferred_element_type=jnp.float32)
    a = a + jnp.concatenate([ba_ref[...], jnp.zeros((1, BN), jnp.float32)], 0)
    a_ref[...] = jnp.concatenate([a, jnp.zeros((6, BN), jnp.float32)], 0)
    for fc in range(FC):
        ht_ref[fc] = h[:, fc * CW:(fc + 1) * CW]


def _out_body(ht_ref, agg_ref, w3_ref, b3_ref, g3_ref, bb3_ref, o_ref):
    acc = jnp.zeros((BN, D), jnp.float32)
    for fc in range(FC):
        z = ht_ref[fc] + 0.5 * agg_ref[fc]
        acc = acc + lax.dot_general(
            z, w3_ref[...][:, fc * CW:(fc + 1) * CW], (((1,), (1,)), ((), ())),
            preferred_element_type=jnp.float32)
    o_ref[...] = _ln(acc + b3_ref[...], g3_ref[...], bb3_ref[...])


def _edge_body(ht_hbm, a_hbm, eidx_hbm, agg_hbm,
               a1_v, a2_v, row_v, col_v, att_v,
               gbuf0, gbuf1, sbuf0, sbuf1, agg_sh,
               gs0, gs1, ss0, ss1):
    c = lax.axis_index("c")
    s = lax.axis_index("s")

    pltpu.sync_copy(a_hbm.at[0], a1_v)
    pltpu.sync_copy(a_hbm.at[1], a2_v)
    pltpu.sync_copy(eidx_hbm.at[0, s], row_v)
    pltpu.sync_copy(eidx_hbm.at[1, s], col_v)

    def att_body(k, _):
        b = k // (EB // 16)
        o = pl.multiple_of((k % (EB // 16)) * 16, 16)
        r16 = row_v[b, pl.ds(o, 16)]
        c16 = col_v[b, pl.ds(o, 16)]
        z = plsc.load_gather(a1_v, [r16]) + plsc.load_gather(a2_v, [c16])
        att_v[b, pl.ds(o, 16)] = 1.0 / (1.0 + jnp.exp(-z))
        return 0
    lax.fori_loop(0, NB * (EB // 16), att_body, 0, unroll=2)

    base = pl.multiple_of(s * STRIPE, EB)
    gb = (gbuf0, gbuf1)
    sb = (sbuf0, sbuf1)
    gs = (gs0, gs1)
    ss = (ss0, ss1)

    def mult(src, dst, b):
        def m16(e16, _):
            o = pl.multiple_of(e16 * 16, 16)
            att16 = att_v[b, pl.ds(o, 16)]
            for r in range(16):
                asp = att16.at[jnp.full((16,), r, jnp.int32)].get(
                    mode="promise_in_bounds")
                aspb = plsc.pack(asp, asp,
                                 format=plsc.PackFormat.INTERLEAVED)
                for v in range(CW // 32):
                    x = plsc.pack(src[o + r, pl.ds(v * 32, 16)],
                                  src[o + r, pl.ds(v * 32 + 16, 16)],
                                  format=plsc.PackFormat.INTERLEAVED)
                    dst[o + r, pl.ds(v * 32, 32)] = x * aspb
            return 0
        lax.fori_loop(0, EB // 16, m16, 0)

    zero32 = jnp.zeros((32,), jnp.bfloat16)

    for j in range(JC):
        fc = c * JC + j
        ht_fc = ht_hbm.at[fc]

        def zrow(i, _):
            for v in range(CW // 32):
                sbuf0[i, pl.ds(v * 32, 32)] = zero32
            return 0
        lax.fori_loop(0, EB, zrow, 0)
        for q in range(STRIPE // EB):
            pltpu.sync_copy(sbuf0, agg_sh.at[pl.ds(base + q * EB, EB)])
        plsc.subcore_barrier()

        pltpu.async_copy(ht_fc.at[col_v.at[0]], gbuf0, gs0)
        pltpu.async_copy(ht_fc.at[col_v.at[1]], gbuf1, gs1)

        def pair(i, _):
            w2 = i * 2
            for b in range(2):
                w = w2 + b
                pltpu.make_async_copy(ht_fc.at[col_v.at[0]], gb[b], gs[b]).wait()

                @pl.when(w2 >= 2)
                def _():
                    pltpu.make_async_copy(sb[b], agg_sh.at[row_v.at[0]],
                                          ss[b]).wait()
                mult(gb[b], sb[b], w)

                @pl.when(w + 2 < NB)
                def _():
                    pltpu.async_copy(ht_fc.at[col_v.at[w + 2]], gb[b], gs[b])
                pltpu.async_copy(sb[b], agg_sh.at[row_v.at[w]], ss[b],
                                 add=True)
            return 0
        lax.fori_loop(0, NB // 2, pair, 0)

        pltpu.make_async_copy(sbuf0, agg_sh.at[row_v.at[0]], ss0).wait()
        pltpu.make_async_copy(sbuf1, agg_sh.at[row_v.at[0]], ss1).wait()

        plsc.subcore_barrier()
        for q in range(STRIPE // EB):
            pltpu.sync_copy(agg_sh.at[pl.ds(base + q * EB, EB)], sbuf0)

            def crow(i, _):
                for v in range(CW // 32):
                    lo, hi = plsc.unpack(sbuf0[i, pl.ds(v * 32, 32)],
                                         format=plsc.PackFormat.INTERLEAVED)
                    gbuf0[i, pl.ds(v * 32, 16)] = lo
                    gbuf0[i, pl.ds(v * 32 + 16, 16)] = hi
                return 0
            lax.fori_loop(0, EB, crow, 0)
            pltpu.sync_copy(gbuf0,
                            agg_hbm.at[fc].at[pl.ds(base + q * EB, EB)])
        plsc.subcore_barrier()


@jax.jit
def kernel(node_ids, edge_index, node_features, emb,
           W0, b0, g0, bb0, W1, b1, g1, bb1, W2, b2, g2, bb2,
           W3, b3, g3, bb3, Wa, ba):
    f32 = jnp.float32
    vspec = lambda bs, im: pl.BlockSpec(bs, im)
    full = lambda shape: pl.BlockSpec(shape, lambda i: tuple(0 for _ in shape))

    wa_t = jnp.concatenate([Wa[:, :H], Wa[:, H:]], axis=0)
    ba_b = jnp.broadcast_to(ba.reshape(1, 1), (1, BN))
    row1 = lambda v: v.reshape(1, -1)

    mlp = pl.pallas_call(
        _mlp_body,
        grid=(NT,),
        in_specs=[
            vspec((BN, D), lambda i: (i, 0)),
            vspec((BN, D), lambda i: (i, 0)),
            full((H, D)), full((1, H)), full((1, H)), full((1, H)),
            full((H, H)), full((1, H)), full((1, H)), full((1, H)),
            full((H, H)), full((1, H)), full((1, H)), full((1, H)),
            full((2, H)), full((1, BN)),
        ],
        out_specs=[
            pl.BlockSpec((FC, BN, CW), lambda i: (0, i, 0)),
            pl.BlockSpec((8, BN), lambda i: (0, i)),
        ],
        out_shape=[
            jax.ShapeDtypeStruct((FC, NP, CW), f32),
            jax.ShapeDtypeStruct((8, NP), f32),
        ],
    )
    ht, a_nodes = mlp(emb, node_features,
                      W0, row1(b0), row1(g0), row1(bb0),
                      W1, row1(b1), row1(g1), row1(bb1),
                      W2, row1(b2), row1(g2), row1(bb2),
                      wa_t, ba_b)

    npad = EP - E
    pad_r = (jnp.arange(npad, dtype=jnp.int32) % (NP - N)) + N
    pad_c = jnp.arange(npad, dtype=jnp.int32) % N
    row_p = jnp.concatenate([edge_index[0], pad_r])
    col_p = jnp.concatenate([edge_index[1], pad_c])
    eidx = jnp.stack([row_p, col_p]).reshape(2, ET, NB, EB)

    edge = pl.kernel(
        _edge_body,
        out_type=jax.ShapeDtypeStruct((FC, NP, CW), f32),
        mesh=plsc.VectorSubcoreMesh(core_axis_name="c", subcore_axis_name="s"),
        compiler_params=pltpu.CompilerParams(needs_layout_passes=False,
                                             use_tc_tiling_on_sc=False),
        scratch_types=[
            pltpu.VMEM((NP,), f32),
            pltpu.VMEM((NP,), f32),
            pltpu.VMEM((NB, EB), jnp.int32),
            pltpu.VMEM((NB, EB), jnp.int32),
            pltpu.VMEM((NB, EB), f32),
            pltpu.VMEM((EB, CW), f32),
            pltpu.VMEM((EB, CW), f32),
            pltpu.VMEM((EB, CW), jnp.bfloat16),
            pltpu.VMEM((EB, CW), jnp.bfloat16),
            pltpu.VMEM_SHARED((NP, CW), jnp.bfloat16),
            pltpu.SemaphoreType.DMA,
            pltpu.SemaphoreType.DMA,
            pltpu.SemaphoreType.DMA,
            pltpu.SemaphoreType.DMA,
        ],
    )
    agg = edge(ht, a_nodes, eidx)

    out = pl.pallas_call(
        _out_body,
        grid=(NT,),
        in_specs=[
            pl.BlockSpec((FC, BN, CW), lambda i: (0, i, 0)),
            pl.BlockSpec((FC, BN, CW), lambda i: (0, i, 0)),
            full((D, H)), full((1, D)), full((1, D)), full((1, D)),
        ],
        out_specs=pl.BlockSpec((BN, D), lambda i: (i, 0)),
        out_shape=jax.ShapeDtypeStruct((N, D), f32),
    )(ht, agg, W3, row1(b3), row1(g3), row1(bb3))
    return out


# speedup vs baseline: 1.3878x; 1.3878x over previous
import functools

import jax
import jax.numpy as jnp
from jax import lax
from jax.experimental import pallas as pl
from jax.experimental.pallas import tpu as pltpu
from jax.experimental.pallas import tpu_sc as plsc

N = 10000
D = 256
H = 512
E = 160000

NP = 10240
FC = 8
CW = 64
BN = 512
NT = NP // BN
ET = 16
EB = 128
NB = 80
EP = ET * NB * EB
STRIPE = NP // ET
JC = FC // 2


def _ln(x, g, b):
    m = jnp.mean(x, axis=-1, keepdims=True)
    xc = x - m
    v = jnp.mean(xc * xc, axis=-1, keepdims=True)
    return xc * lax.rsqrt(v + 1e-5) * g + b


def _mlp_body(emb_ref, nf_ref, w0_ref, b0_ref, g0_ref, bb0_ref,
              w1_ref, b1_ref, g1_ref, bb1_ref,
              w2_ref, b2_ref, g2_ref, bb2_ref,
              wa_ref, ba_ref, ht_ref, a_ref):
    h = emb_ref[...] + nf_ref[...]
    for w_ref, b_ref, g_ref, bb_ref in (
        (w0_ref, b0_ref, g0_ref, bb0_ref),
        (w1_ref, b1_ref, g1_ref, bb1_ref),
        (w2_ref, b2_ref, g2_ref, bb2_ref),
    ):
        x = lax.dot_general(h, w_ref[...], (((1,), (1,)), ((), ())),
                            preferred_element_type=jnp.float32)
        x = _ln(x + b_ref[...], g_ref[...], bb_ref[...])
        h = jnp.maximum(x, 0.0)
    a = lax.dot_general(wa_ref[...], h, (((1,), (1,)), ((), ())),
                        preferred_element_type=jnp.float32)
    a = a + jnp.concatenate([ba_ref[...], jnp.zeros((1, BN), jnp.float32)], 0)
    a_ref[...] = jnp.concatenate([a, jnp.zeros((6, BN), jnp.float32)], 0)
    for fc in range(FC):
        ht_ref[fc] = h[:, fc * CW:(fc + 1) * CW]


def _out_body(ht_ref, agg_ref, w3_ref, b3_ref, g3_ref, bb3_ref, o_ref):
    acc = jnp.zeros((BN, D), jnp.float32)
    for fc in range(FC):
        z = ht_ref[fc] + 0.5 * agg_ref[fc]
        acc = acc + lax.dot_general(
            z, w3_ref[...][:, fc * CW:(fc + 1) * CW], (((1,), (1,)), ((), ())),
            preferred_element_type=jnp.float32)
    o_ref[...] = _ln(acc + b3_ref[...], g3_ref[...], bb3_ref[...])


def _edge_body(ht_hbm, a_hbm, eidx_hbm, agg_hbm,
               a1_v, a2_v, row_v, col_v, att_v,
               gbuf0, gbuf1, sbuf0, sbuf1, agg_sh,
               gs0, gs1, ss0, ss1):
    c = lax.axis_index("c")
    s = lax.axis_index("s")

    pltpu.sync_copy(a_hbm.at[0], a1_v)
    pltpu.sync_copy(a_hbm.at[1], a2_v)
    pltpu.sync_copy(eidx_hbm.at[0, s], row_v)
    pltpu.sync_copy(eidx_hbm.at[1, s], col_v)

    def att_body(k, _):
        b = k // 8
        o = pl.multiple_of((k % 8) * 16, 16)
        r16 = row_v[b, pl.ds(o, 16)]
        c16 = col_v[b, pl.ds(o, 16)]
        z = plsc.load_gather(a1_v, [r16]) + plsc.load_gather(a2_v, [c16])
        att_v[b, pl.ds(o, 16)] = 1.0 / (1.0 + jnp.exp(-z))
        return 0
    lax.fori_loop(0, NB * 8, att_body, 0, unroll=2)

    zero16 = jnp.zeros((16,), jnp.float32)
    base = pl.multiple_of(s * STRIPE, EB)
    gb = (gbuf0, gbuf1)
    sb = (sbuf0, sbuf1)
    gs = (gs0, gs1)
    ss = (ss0, ss1)

    def mult(src, dst, b):
        def m16(e16, _):
            o = pl.multiple_of(e16 * 16, 16)
            att16 = att_v[b, pl.ds(o, 16)]
            for r in range(16):
                asp = att16.at[jnp.full((16,), r, jnp.int32)].get(
                    mode="promise_in_bounds")
                for v in range(CW // 16):
                    sl = pl.ds(v * 16, 16)
                    dst[o + r, sl] = src[o + r, sl] * asp
            return 0
        lax.fori_loop(0, EB // 16, m16, 0)

    for j in range(JC):
        fc = c * JC + j
        ht_fc = ht_hbm.at[fc]

        def zrow(i, _):
            for v in range(CW // 16):
                gbuf0[i, pl.ds(v * 16, 16)] = zero16
            return 0
        lax.fori_loop(0, EB, zrow, 0)
        for q in range(STRIPE // EB):
            pltpu.sync_copy(gbuf0, agg_sh.at[pl.ds(base + q * EB, EB)])
        plsc.subcore_barrier()

        pltpu.async_copy(ht_fc.at[col_v.at[0]], gbuf0, gs0)
        pltpu.async_copy(ht_fc.at[col_v.at[1]], gbuf1, gs1)

        def pair(i, _):
            w2 = i * 2
            for b in range(2):
                w = w2 + b
                pltpu.make_async_copy(ht_fc.at[col_v.at[0]], gb[b], gs[b]).wait()

                @pl.when(w2 >= 2)
                def _():
                    pltpu.make_async_copy(sb[b], agg_sh.at[row_v.at[0]],
                                          ss[b]).wait()
                mult(gb[b], sb[b], w)

                @pl.when(w + 2 < NB)
                def _():
                    pltpu.async_copy(ht_fc.at[col_v.at[w + 2]], gb[b], gs[b])
                pltpu.async_copy(sb[b], agg_sh.at[row_v.at[w]], ss[b],
                                 add=True)
            return 0
        lax.fori_loop(0, NB // 2, pair, 0)

        pltpu.make_async_copy(sbuf0, agg_sh.at[row_v.at[0]], ss0).wait()
        pltpu.make_async_copy(sbuf1, agg_sh.at[row_v.at[0]], ss1).wait()

        plsc.subcore_barrier()
        pltpu.sync_copy(agg_sh.at[pl.ds(base, STRIPE)],
                        agg_hbm.at[fc].at[pl.ds(base, STRIPE)])
        plsc.subcore_barrier()


@jax.jit
def kernel(node_ids, edge_index, node_features, emb,
           W0, b0, g0, bb0, W1, b1, g1, bb1, W2, b2, g2, bb2,
           W3, b3, g3, bb3, Wa, ba):
    f32 = jnp.float32
    vspec = lambda bs, im: pl.BlockSpec(bs, im)
    full = lambda shape: pl.BlockSpec(shape, lambda i: tuple(0 for _ in shape))

    wa_t = jnp.concatenate([Wa[:, :H], Wa[:, H:]], axis=0)
    ba_b = jnp.broadcast_to(ba.reshape(1, 1), (1, BN))
    row1 = lambda v: v.reshape(1, -1)

    mlp = pl.pallas_call(
        _mlp_body,
        grid=(NT,),
        in_specs=[
            vspec((BN, D), lambda i: (i, 0)),
            vspec((BN, D), lambda i: (i, 0)),
            full((H, D)), full((1, H)), full((1, H)), full((1, H)),
            full((H, H)), full((1, H)), full((1, H)), full((1, H)),
            full((H, H)), full((1, H)), full((1, H)), full((1, H)),
            full((2, H)), full((1, BN)),
        ],
        out_specs=[
            pl.BlockSpec((FC, BN, CW), lambda i: (0, i, 0)),
            pl.BlockSpec((8, BN), lambda i: (0, i)),
        ],
        out_shape=[
            jax.ShapeDtypeStruct((FC, NP, CW), f32),
            jax.ShapeDtypeStruct((8, NP), f32),
        ],
    )
    ht, a_nodes = mlp(emb, node_features,
                      W0, row1(b0), row1(g0), row1(bb0),
                      W1, row1(b1), row1(g1), row1(bb1),
                      W2, row1(b2), row1(g2), row1(bb2),
                      wa_t, ba_b)

    npad = EP - E
    pad_r = (jnp.arange(npad, dtype=jnp.int32) % (NP - N)) + N
    pad_c = jnp.arange(npad, dtype=jnp.int32) % N
    row_p = jnp.concatenate([edge_index[0], pad_r])
    col_p = jnp.concatenate([edge_index[1], pad_c])
    eidx = jnp.stack([row_p, col_p]).reshape(2, ET, NB, EB)

    edge = pl.kernel(
        _edge_body,
        out_type=jax.ShapeDtypeStruct((FC, NP, CW), f32),
        mesh=plsc.VectorSubcoreMesh(core_axis_name="c", subcore_axis_name="s"),
        compiler_params=pltpu.CompilerParams(needs_layout_passes=False,
                                             use_tc_tiling_on_sc=False),
        scratch_types=[
            pltpu.VMEM((NP,), f32),
            pltpu.VMEM((NP,), f32),
            pltpu.VMEM((NB, EB), jnp.int32),
            pltpu.VMEM((NB, EB), jnp.int32),
            pltpu.VMEM((NB, EB), f32),
            pltpu.VMEM((EB, CW), f32),
            pltpu.VMEM((EB, CW), f32),
            pltpu.VMEM((EB, CW), f32),
            pltpu.VMEM((EB, CW), f32),
            pltpu.VMEM_SHARED((NP, CW), f32),
            pltpu.SemaphoreType.DMA,
            pltpu.SemaphoreType.DMA,
            pltpu.SemaphoreType.DMA,
            pltpu.SemaphoreType.DMA,
        ],
    )
    agg = edge(ht, a_nodes, eidx)

    out = pl.pallas_call(
        _out_body,
        grid=(NT,),
        in_specs=[
            pl.BlockSpec((FC, BN, CW), lambda i: (0, i, 0)),
            pl.BlockSpec((FC, BN, CW), lambda i: (0, i, 0)),
            full((D, H)), full((1, D)), full((1, D)), full((1, D)),
        ],
        out_specs=pl.BlockSpec((BN, D), lambda i: (i, 0)),
        out_shape=jax.ShapeDtypeStruct((N, D), f32),
    )(ht, agg, W3, row1(b3), row1(g3), row1(bb3))
    return out
